# baseline (speedup 1.0000x reference)
import functools

import jax
import jax.numpy as jnp
from jax import lax
from jax.experimental import pallas as pl
from jax.experimental.pallas import tpu as pltpu
from jax.experimental.pallas import tpu_sc as plsc

_D = 32
_BC = 32
_NW = 32
_LP = 128


@functools.partial(jax.jit, static_argnums=(3, 4))
def _gather_weight(table, idx, w, n_b, n_l):
    bpw = n_b // _NW
    n_b_pad = idx.shape[0]
    n_chunks = bpw // _BC
    chunk_rows = _BC * n_l
    rows_per_w = bpw * n_l
    mesh = plsc.VectorSubcoreMesh(core_axis_name="c", subcore_axis_name="s")

    @functools.partial(
        pl.kernel,
        mesh=mesh,
        out_type=jax.ShapeDtypeStruct((n_b, n_l, _D), jnp.float32),
        compiler_params=pltpu.CompilerParams(use_tc_tiling_on_sc=False),
        scratch_types=[
            pltpu.VMEM((bpw, _LP), jnp.int32),
            pltpu.VMEM((bpw, _LP), jnp.float32),
            pltpu.VMEM((rows_per_w,), jnp.int32),
            pltpu.VMEM((rows_per_w,), jnp.float32),
            pltpu.VMEM((chunk_rows, _D), jnp.float32),
            pltpu.SemaphoreType.DMA,
        ],
    )
    def k(table_hbm, idx_hbm, w_hbm, out_hbm,
          idx2_v, w2_v, idxf_v, wf_v, rows_v, sem):
        wid = lax.axis_index("s") * 2 + lax.axis_index("c")
        b0 = wid * bpw

        pltpu.sync_copy(idx_hbm.at[pl.ds(b0, bpw), :], idx2_v)
        pltpu.sync_copy(w_hbm.at[pl.ds(b0, bpw), :], w2_v)

        starts = (0, 16, 32, n_l - 16)

        def flat_body(b, c):
            base = b * n_l
            for s in starts:
                idxf_v[pl.ds(base + s, 16)] = idx2_v[b, s:s + 16]
                wf_v[pl.ds(base + s, 16)] = w2_v[b, s:s + 16]
            return c

        lax.fori_loop(0, bpw, flat_body, 0)

        def chunk_body(g, carry):
            pltpu.async_copy(
                table_hbm.at[idxf_v.at[pl.ds(g * chunk_rows, chunk_rows)]],
                rows_v, sem,
            ).wait()

            def group_body(g16, c):
                base16 = g16 * 16
                wvec = wf_v[pl.ds(g * chunk_rows + base16, 16)]
                for j in range(16):
                    wb = lax.broadcast(wvec[j], (16,))
                    i = base16 + j
                    rows_v[i, 0:16] = rows_v[i, 0:16] * wb
                    rows_v[i, 16:32] = rows_v[i, 16:32] * wb
                return c

            lax.fori_loop(0, chunk_rows // 16, group_body, 0)

            def out_body(br, c):
                pltpu.sync_copy(
                    rows_v.at[pl.ds(br * n_l, n_l), :],
                    out_hbm.at[b0 + g * _BC + br],
                )
                return c

            lax.fori_loop(0, _BC, out_body, 0)
            return carry

        lax.fori_loop(0, n_chunks, chunk_body, 0)

    return k(table, idx, w)


def kernel(target_indices, target_weights, embedding_weight):
    b, l = target_indices.shape
    pb = 7 * b
    idx_p = jnp.pad(
        target_indices.astype(jnp.int32), ((0, pb), (0, _LP - l))
    )
    w_p = jnp.pad(target_weights, ((0, pb), (0, _LP - l)))
    idx_p, w_p = jax.lax.optimization_barrier((idx_p, w_p))
    return _gather_weight(embedding_weight, idx_p, w_p, b, l)


# speedup vs baseline: 1.0071x; 1.0071x over previous
import functools

import jax
import jax.numpy as jnp
from jax import lax
from jax.experimental import pallas as pl
from jax.experimental.pallas import tpu as pltpu
from jax.experimental.pallas import tpu_sc as plsc

_D = 32
_BC = 32
_NW = 32


@functools.partial(jax.jit, static_argnums=(3, 4))
def _gather_weight(table, idx, w, n_b, n_l):
    bpw = n_b // _NW
    n_chunks = bpw // _BC
    chunk_rows = _BC * n_l
    rows_per_w = bpw * n_l
    mesh = plsc.VectorSubcoreMesh(core_axis_name="c", subcore_axis_name="s")

    @functools.partial(
        pl.kernel,
        mesh=mesh,
        out_type=jax.ShapeDtypeStruct((n_b, n_l, _D), jnp.float32),
        compiler_params=pltpu.CompilerParams(use_tc_tiling_on_sc=False),
        scratch_types=[
            pltpu.VMEM((bpw, n_l), jnp.int32),
            pltpu.VMEM((bpw, n_l), jnp.float32),
            pltpu.VMEM((rows_per_w,), jnp.int32),
            pltpu.VMEM((rows_per_w,), jnp.float32),
            pltpu.VMEM((chunk_rows, _D), jnp.float32),
            pltpu.SemaphoreType.DMA,
        ],
    )
    def k(table_hbm, idx_hbm, w_hbm, out_hbm,
          idx2_v, w2_v, idxf_v, wf_v, rows_v, sem):
        wid = lax.axis_index("s") * 2 + lax.axis_index("c")
        b0 = wid * bpw

        pltpu.sync_copy(idx_hbm.at[pl.ds(b0, bpw), :], idx2_v)
        pltpu.sync_copy(w_hbm.at[pl.ds(b0, bpw), :], w2_v)

        starts = (0, 16, 32, n_l - 16)

        def flat_body(b, c):
            base = b * n_l
            for s in starts:
                idxf_v[pl.ds(base + s, 16)] = idx2_v[b, s:s + 16]
                wf_v[pl.ds(base + s, 16)] = w2_v[b, s:s + 16]
            return c

        lax.fori_loop(0, bpw, flat_body, 0)

        def chunk_body(g, carry):
            pltpu.async_copy(
                table_hbm.at[idxf_v.at[pl.ds(g * chunk_rows, chunk_rows)]],
                rows_v, sem,
            ).wait()

            def group_body(g16, c):
                base16 = g16 * 16
                wvec = wf_v[pl.ds(g * chunk_rows + base16, 16)]
                for j in range(16):
                    wb = lax.broadcast(wvec[j], (16,))
                    i = base16 + j
                    rows_v[i, 0:16] = rows_v[i, 0:16] * wb
                    rows_v[i, 16:32] = rows_v[i, 16:32] * wb
                return c

            lax.fori_loop(0, chunk_rows // 16, group_body, 0)

            def out_body(br, c):
                pltpu.sync_copy(
                    rows_v.at[pl.ds(br * n_l, n_l), :],
                    out_hbm.at[b0 + g * _BC + br],
                )
                return c

            lax.fori_loop(0, _BC, out_body, 0)
            return carry

        lax.fori_loop(0, n_chunks, chunk_body, 0)

    return k(table, idx, w)


def kernel(target_indices, target_weights, embedding_weight):
    b, l = target_indices.shape
    return _gather_weight(
        embedding_weight, target_indices.astype(jnp.int32), target_weights, b, l
    )


# speedup vs baseline: 1.0326x; 1.0253x over previous
import functools

import jax
import jax.numpy as jnp
from jax import lax
from jax.experimental import pallas as pl
from jax.experimental.pallas import tpu as pltpu
from jax.experimental.pallas import tpu_sc as plsc

_D = 32
_BC = 32
_NW = 32


@functools.partial(jax.jit, static_argnums=(3, 4))
def _gather_weight(table, idx, w, n_b, n_l):
    bpw = n_b // _NW
    n_chunks = bpw // _BC
    chunk_rows = _BC * n_l
    rows_per_w = bpw * n_l
    mesh = plsc.VectorSubcoreMesh(core_axis_name="c", subcore_axis_name="s")

    @functools.partial(
        pl.kernel,
        mesh=mesh,
        out_type=jax.ShapeDtypeStruct((n_b, n_l, _D), jnp.float32),
        compiler_params=pltpu.CompilerParams(use_tc_tiling_on_sc=False),
        scratch_types=[
            pltpu.VMEM((bpw, n_l), jnp.int32),
            pltpu.VMEM((bpw, n_l), jnp.float32),
            pltpu.VMEM((rows_per_w,), jnp.int32),
            pltpu.VMEM((rows_per_w,), jnp.float32),
            pltpu.VMEM((chunk_rows, _D), jnp.float32),
            pltpu.VMEM((chunk_rows, _D), jnp.float32),
            pltpu.SemaphoreType.DMA,
            pltpu.SemaphoreType.DMA,
            pltpu.SemaphoreType.DMA,
            pltpu.SemaphoreType.DMA,
        ],
    )
    def k(table_hbm, idx_hbm, w_hbm, out_hbm,
          idx2_v, w2_v, idxf_v, wf_v, rows_a, rows_b,
          sg_a, sg_b, so_a, so_b):
        wid = lax.axis_index("s") * 2 + lax.axis_index("c")
        b0 = wid * bpw

        pltpu.sync_copy(idx_hbm.at[pl.ds(b0, bpw), :], idx2_v)
        pltpu.sync_copy(w_hbm.at[pl.ds(b0, bpw), :], w2_v)

        starts = (0, 16, 32, n_l - 16)

        def flat_body(b, c):
            base = b * n_l
            for s in starts:
                idxf_v[pl.ds(base + s, 16)] = idx2_v[b, s:s + 16]
                wf_v[pl.ds(base + s, 16)] = w2_v[b, s:s + 16]
            return c

        lax.fori_loop(0, bpw, flat_body, 0)

        bufs = (rows_a, rows_b)
        gsems = (sg_a, sg_b)
        osems = (so_a, so_b)

        def fire_gather(g):
            return pltpu.async_copy(
                table_hbm.at[idxf_v.at[pl.ds(g * chunk_rows, chunk_rows)]],
                bufs[g % 2], gsems[g % 2],
            )

        def multiply(g):
            rows_v = bufs[g % 2]

            def group_body(g16, c):
                base16 = g16 * 16
                wvec = wf_v[pl.ds(g * chunk_rows + base16, 16)]
                for j in range(16):
                    wb = lax.broadcast(wvec[j], (16,))
                    i = base16 + j
                    rows_v[i, 0:16] = rows_v[i, 0:16] * wb
                    rows_v[i, 16:32] = rows_v[i, 16:32] * wb
                return c

            lax.fori_loop(0, chunk_rows // 16, group_body, 0)

        def fire_outs(g):
            rows_v = bufs[g % 2]
            return [
                pltpu.async_copy(
                    rows_v.at[pl.ds(br * n_l, n_l), :],
                    out_hbm.at[b0 + g * _BC + br],
                    osems[g % 2],
                )
                for br in range(_BC)
            ]

        gathers = [None] * n_chunks
        outs = [None] * n_chunks
        gathers[0] = fire_gather(0)
        for g in range(n_chunks):
            gathers[g].wait()
            if g + 1 < n_chunks:
                if g >= 1:
                    for h in outs[g - 1]:
                        h.wait()
                gathers[g + 1] = fire_gather(g + 1)
            multiply(g)
            outs[g] = fire_outs(g)
        for g in (n_chunks - 2, n_chunks - 1):
            for h in outs[g]:
                h.wait()

    return k(table, idx, w)


def kernel(target_indices, target_weights, embedding_weight):
    b, l = target_indices.shape
    return _gather_weight(
        embedding_weight, target_indices.astype(jnp.int32), target_weights, b, l
    )
